# Initial kernel scaffold; baseline (speedup 1.0000x reference)
import jax
import jax.numpy as jnp
from jax.experimental import pallas as pl


def kernel(edge_index, x, edge_weight, W_gcn, b_gcn, bn_gamma, bn_beta, bn_mean, bn_var, W_ih, W_hh, b_ih, b_hh, W_fc, b_fc):
    raise NotImplementedError("write your pallas kernel here")


# speedup vs baseline: 13.5530x; 13.5530x over previous
import functools

import jax
import jax.numpy as jnp
from jax import lax
from jax.experimental import pallas as pl
from jax.experimental.pallas import tpu as pltpu
from jax.experimental.pallas import tpu_sc as plsc

_N = 10000
_E = 320000
_DI = 128
_H = 256
_G4 = 4 * _H
_DO = 128

_NW = 32
_CHUNK = 128
_C = 79
_EPT = _C * _CHUNK
_EPAD = _NW * _EPT
_NPAD = 10240
_NPT = _NPAD // 16

_TB = 400
_NB = _N // _TB

_sc_mesh = plsc.VectorSubcoreMesh(core_axis_name="c", subcore_axis_name="s")


@functools.partial(
    pl.kernel,
    out_type=jax.ShapeDtypeStruct((2, _NPAD), jnp.float32),
    mesh=_sc_mesh,
    scratch_types=[
        pltpu.VMEM((_C, _CHUNK), jnp.int32),
        pltpu.VMEM((_C, _CHUNK), jnp.float32),
        pltpu.VMEM_SHARED((_NPAD,), jnp.float32),
        pltpu.SemaphoreType.DMA,
    ],
)
def _sc_deg(col_hbm, w_hbm, zeros_hbm, out_hbm, col_v, w_v, deg_sh, sem):
    cid = lax.axis_index("c")
    sid = lax.axis_index("s")
    wid = sid * 2 + cid
    base = sid * _NPT
    pltpu.sync_copy(zeros_hbm.at[pl.ds(base, _NPT)], deg_sh.at[pl.ds(base, _NPT)])
    plsc.subcore_barrier()
    pltpu.sync_copy(col_hbm.at[wid], col_v)
    pltpu.sync_copy(w_hbm.at[wid], w_v)

    def body(j, carry):
        pltpu.sync_copy(w_v.at[j], deg_sh.at[col_v.at[j]], add=True)
        return carry

    lax.fori_loop(0, _C, body, 0)
    plsc.subcore_barrier()
    pltpu.sync_copy(deg_sh.at[pl.ds(base, _NPT)], out_hbm.at[cid, pl.ds(base, _NPT)])


@functools.partial(
    pl.kernel,
    out_type=jax.ShapeDtypeStruct((2, _NPAD, _DI), jnp.float32),
    mesh=_sc_mesh,
    scratch_types=[
        pltpu.VMEM((_C, _CHUNK), jnp.int32),
        pltpu.VMEM((_C, _CHUNK), jnp.int32),
        pltpu.VMEM((_C, _CHUNK), jnp.float32),
        pltpu.VMEM((_CHUNK, _DI), jnp.float32),
        pltpu.VMEM_SHARED((_NPAD, _DI), jnp.float32),
        pltpu.SemaphoreType.DMA,
    ],
)
def _sc_msg(row_hbm, col_hbm, w_hbm, z_hbm, zeros_hbm, out_hbm,
            row_v, col_v, w_v, rows_v, agg_sh, sem):
    cid = lax.axis_index("c")
    sid = lax.axis_index("s")
    wid = sid * 2 + cid
    base = sid * _NPT
    pltpu.sync_copy(zeros_hbm.at[pl.ds(base, _NPT)], agg_sh.at[pl.ds(base, _NPT)])
    plsc.subcore_barrier()
    pltpu.sync_copy(row_hbm.at[wid], row_v)
    pltpu.sync_copy(col_hbm.at[wid], col_v)
    pltpu.sync_copy(w_hbm.at[wid], w_v)

    def chunk(j, carry):
        pltpu.async_copy(z_hbm.at[row_v.at[j]], rows_v, sem).wait()

        def group(k, c2):
            nv = w_v[j, pl.ds(k * 16, 16)]
            for l in range(16):
                s = nv[l]
                e = k * 16 + l
                for q in range(_DI // 16):
                    sl = pl.ds(q * 16, 16)
                    rows_v[e, sl] = rows_v[e, sl] * s
            return c2

        lax.fori_loop(0, _CHUNK // 16, group, 0)
        pltpu.sync_copy(rows_v, agg_sh.at[col_v.at[j]], add=True)
        return carry

    lax.fori_loop(0, _C, chunk, 0)
    plsc.subcore_barrier()
    pltpu.sync_copy(agg_sh.at[pl.ds(base, _NPT)], out_hbm.at[cid, pl.ds(base, _NPT)])


def _prep_body(deg_ref, x_ref, dinv_ref, z_ref):
    deg = deg_ref[0] + deg_ref[1] + 1.0
    dinv = lax.rsqrt(deg)
    dinv_ref[...] = dinv
    z_ref[...] = dinv * x_ref[...]


def _prep(deg_r, x):
    return pl.pallas_call(
        _prep_body,
        out_shape=(
            jax.ShapeDtypeStruct((_N, 1), jnp.float32),
            jax.ShapeDtypeStruct((_N, _DI), jnp.float32),
        ),
    )(deg_r, x)


def _mid_body(b0_ref, b1_ref, z_ref, dinv_ref, wg_ref, bg_ref, gam_ref, bet_ref,
              mu_ref, var_ref, wih_ref, bih_ref, bhh_ref, p_ref):
    agg = (b0_ref[...] + b1_ref[...] + z_ref[...]) * dinv_ref[...]
    h = jnp.dot(agg, wg_ref[...], preferred_element_type=jnp.float32) + bg_ref[...]
    h = jnp.maximum(h, 0.0)
    scale = gam_ref[...] * lax.rsqrt(var_ref[...] + 1e-5)
    hbn = (h - mu_ref[...]) * scale + bet_ref[...]
    p_ref[...] = (jnp.dot(hbn, wih_ref[...], preferred_element_type=jnp.float32)
                  + bih_ref[...] + bhh_ref[...])


def _mid(b0, b1, z, dinv, wg_t, bg, gam, bet, mu, var, wih_t, bih, bhh):
    row_spec = pl.BlockSpec((_TB, _DI), lambda i: (i, 0))
    full = lambda shape: pl.BlockSpec(shape, lambda i: tuple(0 for _ in shape))
    return pl.pallas_call(
        _mid_body,
        grid=(_NB,),
        in_specs=[
            row_spec, row_spec, row_spec,
            pl.BlockSpec((_TB, 1), lambda i: (i, 0)),
            full((_DI, _H)), full((1, _H)), full((1, _H)), full((1, _H)),
            full((1, _H)), full((1, _H)),
            full((_H, _G4)), full((1, _G4)), full((1, _G4)),
        ],
        out_specs=pl.BlockSpec((_TB, _G4), lambda i: (i, 0)),
        out_shape=jax.ShapeDtypeStruct((_N, _G4), jnp.float32),
    )(b0, b1, z, dinv, wg_t, bg, gam, bet, mu, var, wih_t, bih, bhh)


def _lstm_body(p_ref, whhb_ref, wfc_ref, bfc_ref, y_ref, h_s, c_s, seq_s):
    pid = pl.program_id(0)

    @pl.when(pid == 0)
    def _():
        h_s[...] = jnp.zeros_like(h_s)
        c_s[...] = jnp.zeros_like(c_s)

    whh = whhb_ref[...]

    def step(t, carry):
        hp, cp = carry
        hb = hp.astype(jnp.bfloat16)
        gates = p_ref[pl.ds(t, 1), :] + jnp.dot(
            hb, whh, preferred_element_type=jnp.float32)
        i = jax.nn.sigmoid(gates[:, 0:_H])
        f = jax.nn.sigmoid(gates[:, _H:2 * _H])
        g = jnp.tanh(gates[:, 2 * _H:3 * _H])
        o = jax.nn.sigmoid(gates[:, 3 * _H:4 * _H])
        c = f * cp + i * g
        h = o * jnp.tanh(c)
        seq_s[pl.ds(t, 1), :] = h
        return (h, c)

    h_fin, c_fin = lax.fori_loop(0, _TB, step, (h_s[...], c_s[...]),
                                 unroll=4)
    h_s[...] = h_fin
    c_s[...] = c_fin
    y = jnp.dot(seq_s[...], wfc_ref[...], preferred_element_type=jnp.float32) + bfc_ref[...]
    y_ref[...] = jnp.maximum(y, 0.0)


def _lstm(p, whh_b, wfc_t, bfc):
    return pl.pallas_call(
        _lstm_body,
        grid=(_NB,),
        in_specs=[
            pl.BlockSpec((_TB, _G4), lambda i: (i, 0)),
            pl.BlockSpec((_H, _G4), lambda i: (0, 0)),
            pl.BlockSpec((_H, _DO), lambda i: (0, 0)),
            pl.BlockSpec((1, _DO), lambda i: (0, 0)),
        ],
        out_specs=pl.BlockSpec((_TB, _DO), lambda i: (i, 0)),
        out_shape=jax.ShapeDtypeStruct((_N, _DO), jnp.float32),
        scratch_shapes=[
            pltpu.VMEM((1, _H), jnp.float32),
            pltpu.VMEM((1, _H), jnp.float32),
            pltpu.VMEM((_TB, _H), jnp.float32),
        ],
    )(p, whh_b, wfc_t, bfc)


def kernel(edge_index, x, edge_weight, W_gcn, b_gcn, bn_gamma, bn_beta,
           bn_mean, bn_var, W_ih, W_hh, b_ih, b_hh, W_fc, b_fc):
    row = edge_index[0]
    col = edge_index[1]
    pad = _EPAD - _E
    rowp = jnp.concatenate([row, jnp.zeros((pad,), row.dtype)]).reshape(_NW, _C, _CHUNK)
    colp = jnp.concatenate([col, jnp.zeros((pad,), col.dtype)]).reshape(_NW, _C, _CHUNK)
    wp = jnp.concatenate([edge_weight, jnp.zeros((pad,), edge_weight.dtype)]).reshape(_NW, _C, _CHUNK)
    zeros1 = jnp.zeros((_NPAD,), jnp.float32)
    zeros2 = jnp.zeros((_NPAD, _DI), jnp.float32)

    deg2 = _sc_deg(colp, wp, zeros1)
    deg_r = deg2[:, :_N].reshape(2, _N, 1)
    dinv, z = _prep(deg_r, x)
    buf = _sc_msg(rowp, colp, wp, z, zeros2)

    p = _mid(buf[0, :_N], buf[1, :_N], z, dinv,
             W_gcn.T, b_gcn.reshape(1, _H), bn_gamma.reshape(1, _H),
             bn_beta.reshape(1, _H), bn_mean.reshape(1, _H), bn_var.reshape(1, _H),
             W_ih.T, b_ih.reshape(1, _G4), b_hh.reshape(1, _G4))
    y = _lstm(p, W_hh.T.astype(jnp.bfloat16), W_fc.T, b_fc.reshape(1, _DO))
    return y
